# Initial kernel scaffold; baseline (speedup 1.0000x reference)
import functools

import jax
import jax.numpy as jnp
from jax import lax
from jax.experimental import pallas as pl
from jax.experimental.pallas import tpu as pltpu
from jax.experimental.pallas import tpu_sc as plsc

N = 10000
E = 320000
D = 128
DE = 16
ALPHA = 0.5

_NC = 2
_NS = 16
_NW = _NC * _NS
_CH = 128
_KPW = 79
_E_PAD = _NW * _KPW * _CH
_K_REAL = E // _CH
_RPS = N // _NS


def _proj_body(x_ref, w_ref, ps_ref, pd_ref):
    p = jnp.dot(x_ref[...], w_ref[...], preferred_element_type=jnp.float32)
    ps_ref[...] = p[:, :DE]
    pd_ref[...] = p[:, DE:]


def _tc_proj(x, wsd):
    return pl.pallas_call(
        _proj_body,
        grid=(10,),
        in_specs=[
            pl.BlockSpec((N // 10, D), lambda i: (i, 0)),
            pl.BlockSpec((D, 2 * DE), lambda i: (0, 0)),
        ],
        out_specs=[
            pl.BlockSpec((N // 10, DE), lambda i: (i, 0)),
            pl.BlockSpec((N // 10, DE), lambda i: (i, 0)),
        ],
        out_shape=[jax.ShapeDtypeStruct((N, DE), jnp.float32)] * 2,
    )(x, wsd)


def _base_body(ea_ref, w_ref, b_ref, o_ref):
    o_ref[...] = (
        jnp.dot(ea_ref[...], w_ref[...], preferred_element_type=jnp.float32)
        + b_ref[...]
    )


def _tc_base(ea, wa, be2d):
    blk = E // 20
    return pl.pallas_call(
        _base_body,
        grid=(20,),
        in_specs=[
            pl.BlockSpec((blk, DE), lambda i: (i, 0)),
            pl.BlockSpec((DE, DE), lambda i: (0, 0)),
            pl.BlockSpec((1, DE), lambda i: (0, 0)),
        ],
        out_specs=pl.BlockSpec((blk, DE), lambda i: (i, 0)),
        out_shape=jax.ShapeDtypeStruct((E, DE), jnp.float32),
    )(ea, wa, be2d)


def _node_body(with_proj, x_ref, agg_ref, wnx_ref, wna_ref, bn_ref, wsd_ref,
               *out_refs):
    agg = agg_ref[0] + agg_ref[1]
    dx = (
        jnp.dot(x_ref[...], wnx_ref[...], preferred_element_type=jnp.float32)
        + jnp.dot(agg, wna_ref[...], preferred_element_type=jnp.float32)
        + bn_ref[...]
    )
    sa = jnp.float32(ALPHA) ** 0.5
    sb = jnp.float32(1.0 - ALPHA) ** 0.5
    xn = sa * jnp.maximum(dx, 0.0) + sb * x_ref[...]
    out_refs[0][...] = xn
    if with_proj:
        p = jnp.dot(xn, wsd_ref[...], preferred_element_type=jnp.float32)
        out_refs[1][...] = p[:, :DE]
        out_refs[2][...] = p[:, DE:]


def _tc_node(x, aggp, wnx, wna, bn2d, wsd_next, with_proj):
    blk = N // 10
    out_specs = [pl.BlockSpec((blk, D), lambda i: (i, 0))]
    out_shape = [jax.ShapeDtypeStruct((N, D), jnp.float32)]
    if with_proj:
        out_specs += [pl.BlockSpec((blk, DE), lambda i: (i, 0))] * 2
        out_shape += [jax.ShapeDtypeStruct((N, DE), jnp.float32)] * 2
    return pl.pallas_call(
        functools.partial(_node_body, with_proj),
        grid=(10,),
        in_specs=[
            pl.BlockSpec((blk, D), lambda i: (i, 0)),
            pl.BlockSpec((2, blk, DE), lambda i: (0, i, 0)),
            pl.BlockSpec((D, D), lambda i: (0, 0)),
            pl.BlockSpec((DE, D), lambda i: (0, 0)),
            pl.BlockSpec((1, D), lambda i: (0, 0)),
            pl.BlockSpec((D, 2 * DE), lambda i: (0, 0)),
        ],
        out_specs=out_specs,
        out_shape=out_shape,
    )(x, aggp, wnx, wna, bn2d, wsd_next)


def _sc_edge_body(ps_hbm, pd_hbm, base_hbm, srcm_hbm, dstm_hbm, zeros_hbm,
                  ea_hbm, agg_hbm,
                  idx_s, idx_d, rows_s, rows_d, base_v, agg_sh,
                  sem_s, sem_d, sem_b):
    cid = lax.axis_index("c")
    sid = lax.axis_index("s")
    wid = sid * _NC + cid

    pltpu.sync_copy(zeros_hbm.at[pl.ds(sid * _RPS, _RPS)],
                    agg_sh.at[pl.ds(sid * _RPS, _RPS)])
    pltpu.sync_copy(srcm_hbm.at[pl.ds(wid * _KPW, _KPW)], idx_s)
    pltpu.sync_copy(dstm_hbm.at[pl.ds(wid * _KPW, _KPW)], idx_d)
    plsc.subcore_barrier()

    k_hi = jnp.clip(_K_REAL - wid * _KPW, 0, _KPW)

    def chunk(j, carry):
        e0 = (wid * _KPW + j) * _CH
        g1 = pltpu.async_copy(ps_hbm.at[idx_s.at[j]], rows_s, sem_s)
        g2 = pltpu.async_copy(pd_hbm.at[idx_d.at[j]], rows_d, sem_d)
        g3 = pltpu.async_copy(base_hbm.at[pl.ds(e0, _CH)], base_v, sem_b)
        g1.wait()
        g2.wait()
        g3.wait()

        def row(i, c):
            rows_s[i] = jnp.maximum(rows_s[i] + rows_d[i] + base_v[i], 0.0)
            return c

        lax.fori_loop(0, _CH, row, None, unroll=4)
        pltpu.sync_copy(rows_s, ea_hbm.at[pl.ds(e0, _CH)])
        pltpu.sync_copy(rows_s, agg_sh.at[idx_d.at[j]], add=True)
        return carry

    lax.fori_loop(0, k_hi, chunk, None)
    plsc.subcore_barrier()
    pltpu.sync_copy(
        agg_sh.at[pl.ds(sid * _RPS, _RPS)],
        agg_hbm.at[pl.ds(cid * N + sid * _RPS, _RPS)])


_sc_edge = functools.partial(
    pl.kernel,
    out_type=[
        jax.ShapeDtypeStruct((E, DE), jnp.float32),
        jax.ShapeDtypeStruct((2 * N, DE), jnp.float32),
    ],
    mesh=plsc.VectorSubcoreMesh(core_axis_name="c", subcore_axis_name="s"),
    scratch_types=[
        pltpu.VMEM((_KPW, _CH), jnp.int32),
        pltpu.VMEM((_KPW, _CH), jnp.int32),
        pltpu.VMEM((_CH, DE), jnp.float32),
        pltpu.VMEM((_CH, DE), jnp.float32),
        pltpu.VMEM((_CH, DE), jnp.float32),
        pltpu.VMEM_SHARED((N, DE), jnp.float32),
        pltpu.SemaphoreType.DMA,
        pltpu.SemaphoreType.DMA,
        pltpu.SemaphoreType.DMA,
    ],
)(_sc_edge_body)


def kernel(x, edge_index, edge_attr, We1, be1, Wn1, bn1, We2, be2, Wn2, bn2):
    src = edge_index[0]
    dst = edge_index[1]
    pad = _E_PAD - E
    srcm = jnp.pad(src, (0, pad)).reshape(_E_PAD // _CH, _CH)
    dstm = jnp.pad(dst, (0, pad)).reshape(_E_PAD // _CH, _CH)
    zeros = jnp.zeros((N, DE), jnp.float32)

    wsd1 = jnp.concatenate([We1[:D], We1[D:2 * D]], axis=1)
    wsd2 = jnp.concatenate([We2[:D], We2[D:2 * D]], axis=1)
    wa1, wa2 = We1[2 * D:], We2[2 * D:]
    be1_2d, be2_2d = be1[None, :], be2[None, :]
    wnx1, wna1 = Wn1[:D], Wn1[D:]
    wnx2, wna2 = Wn2[:D], Wn2[D:]
    bn1_2d, bn2_2d = bn1[None, :], bn2[None, :]

    ps1, pd1 = _tc_proj(x, wsd1)
    base1 = _tc_base(edge_attr, wa1, be1_2d)
    ea1, aggf1 = _sc_edge(ps1, pd1, base1, srcm, dstm, zeros)
    aggp1 = aggf1.reshape(2, N, DE)
    x2, ps2, pd2 = _tc_node(x, aggp1, wnx1, wna1, bn1_2d, wsd2, True)

    base2 = _tc_base(ea1, wa2, be2_2d)
    ea2, aggf2 = _sc_edge(ps2, pd2, base2, srcm, dstm, zeros)
    aggp2 = aggf2.reshape(2, N, DE)
    (x3,) = _tc_node(x2, aggp2, wnx2, wna2, bn2_2d, wsd2, False)

    return x3, ea2, jnp.concatenate([edge_attr, ea1, ea2], axis=1)


# speedup vs baseline: 3.7144x; 3.7144x over previous
import functools

import jax
import jax.numpy as jnp
from jax import lax
from jax.experimental import pallas as pl
from jax.experimental.pallas import tpu as pltpu
from jax.experimental.pallas import tpu_sc as plsc

N = 10000
E = 320000
D = 128
DE = 16
ALPHA = 0.5

_NC = 2
_NS = 16
_NW = _NC * _NS
_CH = 128
_KPW = 80
_E_PAD = _NW * _KPW * _CH
_K_REAL = E // _CH
_N_PAD = 10240
_RPS = _N_PAD // _NS


def _proj_body(x_ref, w_ref, ps_ref, pd_ref):
    p = jnp.dot(x_ref[...], w_ref[...], preferred_element_type=jnp.float32)
    ps_ref[...] = p[:, :DE]
    pd_ref[...] = p[:, DE:]


def _tc_proj(x, wsd):
    return pl.pallas_call(
        _proj_body,
        grid=(10,),
        in_specs=[
            pl.BlockSpec((N // 10, D), lambda i: (i, 0)),
            pl.BlockSpec((D, 2 * DE), lambda i: (0, 0)),
        ],
        out_specs=[
            pl.BlockSpec((N // 10, DE), lambda i: (i, 0)),
            pl.BlockSpec((N // 10, DE), lambda i: (i, 0)),
        ],
        out_shape=[jax.ShapeDtypeStruct((N, DE), jnp.float32)] * 2,
    )(x, wsd)


def _base_body(ea_ref, w_ref, b_ref, o_ref):
    o_ref[...] = (
        jnp.dot(ea_ref[...], w_ref[...], preferred_element_type=jnp.float32)
        + b_ref[...]
    )


def _tc_base(ea, wa, be2d):
    blk = E // 20
    return pl.pallas_call(
        _base_body,
        grid=(20,),
        in_specs=[
            pl.BlockSpec((blk, DE), lambda i: (i, 0)),
            pl.BlockSpec((DE, DE), lambda i: (0, 0)),
            pl.BlockSpec((1, DE), lambda i: (0, 0)),
        ],
        out_specs=pl.BlockSpec((blk, DE), lambda i: (i, 0)),
        out_shape=jax.ShapeDtypeStruct((E, DE), jnp.float32),
    )(ea, wa, be2d)


def _node_body(with_proj, x_ref, agg_ref, wnx_ref, wna_ref, bn_ref, wsd_ref,
               *out_refs):
    agg = agg_ref[0] + agg_ref[1]
    dx = (
        jnp.dot(x_ref[...], wnx_ref[...], preferred_element_type=jnp.float32)
        + jnp.dot(agg, wna_ref[...], preferred_element_type=jnp.float32)
        + bn_ref[...]
    )
    sa = jnp.float32(ALPHA) ** 0.5
    sb = jnp.float32(1.0 - ALPHA) ** 0.5
    xn = sa * jnp.maximum(dx, 0.0) + sb * x_ref[...]
    out_refs[0][...] = xn
    if with_proj:
        p = jnp.dot(xn, wsd_ref[...], preferred_element_type=jnp.float32)
        out_refs[1][...] = p[:, :DE]
        out_refs[2][...] = p[:, DE:]


def _tc_node(x, aggp, wnx, wna, bn2d, wsd_next, with_proj):
    blk = N // 10
    out_specs = [pl.BlockSpec((blk, D), lambda i: (i, 0))]
    out_shape = [jax.ShapeDtypeStruct((N, D), jnp.float32)]
    if with_proj:
        out_specs += [pl.BlockSpec((blk, DE), lambda i: (i, 0))] * 2
        out_shape += [jax.ShapeDtypeStruct((N, DE), jnp.float32)] * 2
    return pl.pallas_call(
        functools.partial(_node_body, with_proj),
        grid=(10,),
        in_specs=[
            pl.BlockSpec((blk, D), lambda i: (i, 0)),
            pl.BlockSpec((2, blk, DE), lambda i: (0, i, 0)),
            pl.BlockSpec((D, D), lambda i: (0, 0)),
            pl.BlockSpec((DE, D), lambda i: (0, 0)),
            pl.BlockSpec((1, D), lambda i: (0, 0)),
            pl.BlockSpec((D, 2 * DE), lambda i: (0, 0)),
        ],
        out_specs=out_specs,
        out_shape=out_shape,
    )(x, aggp, wnx, wna, bn2d, wsd_next)


def _sc_edge_body(ps_hbm, pd_hbm, base_hbm, srcm_hbm, dstm_hbm, zeros_hbm,
                  ea_hbm, agg_hbm,
                  idx_s, idx_d, rows_s, rows_d, base_v, agg_sh,
                  sem_s, sem_d, sem_b):
    cid = lax.axis_index("c")
    sid = lax.axis_index("s")
    wid = sid * _NC + cid

    pltpu.sync_copy(zeros_hbm.at[pl.ds(sid * _RPS, _RPS)],
                    agg_sh.at[pl.ds(sid * _RPS, _RPS)])
    pltpu.sync_copy(srcm_hbm.at[pl.ds(wid * _KPW, _KPW)], idx_s)
    pltpu.sync_copy(dstm_hbm.at[pl.ds(wid * _KPW, _KPW)], idx_d)
    plsc.subcore_barrier()

    k_hi = jnp.clip(_K_REAL - wid * _KPW, 0, _KPW)

    def chunk(j, carry):
        e0 = (wid * _KPW + j) * _CH
        g1 = pltpu.async_copy(ps_hbm.at[idx_s.at[j]], rows_s, sem_s)
        g2 = pltpu.async_copy(pd_hbm.at[idx_d.at[j]], rows_d, sem_d)
        g3 = pltpu.async_copy(base_hbm.at[pl.ds(e0, _CH)], base_v, sem_b)
        g1.wait()
        g2.wait()
        g3.wait()

        def row(i, c):
            rows_s[i] = jnp.maximum(rows_s[i] + rows_d[i] + base_v[i], 0.0)
            return c

        lax.fori_loop(0, _CH, row, None, unroll=4)
        pltpu.sync_copy(rows_s, ea_hbm.at[pl.ds(e0, _CH)])
        pltpu.sync_copy(rows_s, agg_sh.at[idx_d.at[j]], add=True)
        return carry

    lax.fori_loop(0, k_hi, chunk, None)
    plsc.subcore_barrier()
    pltpu.sync_copy(
        agg_sh.at[pl.ds(sid * _RPS, _RPS)],
        agg_hbm.at[pl.ds(cid * _N_PAD + sid * _RPS, _RPS)])


_sc_edge = functools.partial(
    pl.kernel,
    out_type=[
        jax.ShapeDtypeStruct((E, DE), jnp.float32),
        jax.ShapeDtypeStruct((2 * _N_PAD, DE), jnp.float32),
    ],
    mesh=plsc.VectorSubcoreMesh(core_axis_name="c", subcore_axis_name="s"),
    compiler_params=pltpu.CompilerParams(use_tc_tiling_on_sc=False),
    scratch_types=[
        pltpu.VMEM((_KPW, _CH), jnp.int32),
        pltpu.VMEM((_KPW, _CH), jnp.int32),
        pltpu.VMEM((_CH, DE), jnp.float32),
        pltpu.VMEM((_CH, DE), jnp.float32),
        pltpu.VMEM((_CH, DE), jnp.float32),
        pltpu.VMEM_SHARED((_N_PAD, DE), jnp.float32),
        pltpu.SemaphoreType.DMA,
        pltpu.SemaphoreType.DMA,
        pltpu.SemaphoreType.DMA,
    ],
)(_sc_edge_body)


def kernel(x, edge_index, edge_attr, We1, be1, Wn1, bn1, We2, be2, Wn2, bn2):
    src = edge_index[0]
    dst = edge_index[1]
    pad = _E_PAD - E
    srcm = jnp.pad(src, (0, pad)).reshape(_E_PAD // _CH, _CH)
    dstm = jnp.pad(dst, (0, pad)).reshape(_E_PAD // _CH, _CH)
    zeros = jnp.zeros((_N_PAD, DE), jnp.float32)

    wsd1 = jnp.concatenate([We1[:D], We1[D:2 * D]], axis=1)
    wsd2 = jnp.concatenate([We2[:D], We2[D:2 * D]], axis=1)
    wa1, wa2 = We1[2 * D:], We2[2 * D:]
    be1_2d, be2_2d = be1[None, :], be2[None, :]
    wnx1, wna1 = Wn1[:D], Wn1[D:]
    wnx2, wna2 = Wn2[:D], Wn2[D:]
    bn1_2d, bn2_2d = bn1[None, :], bn2[None, :]

    ps1, pd1 = _tc_proj(x, wsd1)
    base1 = _tc_base(edge_attr, wa1, be1_2d)
    ea1, aggf1 = _sc_edge(ps1, pd1, base1, srcm, dstm, zeros)
    aggp1 = aggf1.reshape(2, _N_PAD, DE)
    x2, ps2, pd2 = _tc_node(x, aggp1, wnx1, wna1, bn1_2d, wsd2, True)

    base2 = _tc_base(ea1, wa2, be2_2d)
    ea2, aggf2 = _sc_edge(ps2, pd2, base2, srcm, dstm, zeros)
    aggp2 = aggf2.reshape(2, _N_PAD, DE)
    (x3,) = _tc_node(x2, aggp2, wnx2, wna2, bn2_2d, wsd2, False)

    return x3, ea2, jnp.concatenate([edge_attr, ea1, ea2], axis=1)


# speedup vs baseline: 3.9302x; 1.0581x over previous
import functools

import jax
import jax.numpy as jnp
from jax import lax
from jax.experimental import pallas as pl
from jax.experimental.pallas import tpu as pltpu
from jax.experimental.pallas import tpu_sc as plsc

N = 10000
E = 320000
D = 128
DE = 16
ALPHA = 0.5

_NC = 2
_NS = 16
_NW = _NC * _NS
_CH = 500
_KPW = 20
_N_PAD = 10240
_RPS = _N_PAD // _NS


def _proj_body(x_ref, w_ref, ps_ref, pd_ref):
    p = jnp.dot(x_ref[...], w_ref[...], preferred_element_type=jnp.float32)
    ps_ref[...] = p[:, :DE]
    pd_ref[...] = p[:, DE:]


def _tc_proj(x, wsd):
    blk = N // 10
    return pl.pallas_call(
        _proj_body,
        grid=(10,),
        in_specs=[
            pl.BlockSpec((blk, D), lambda i: (i, 0)),
            pl.BlockSpec((D, 2 * DE), lambda i: (0, 0)),
        ],
        out_specs=[
            pl.BlockSpec((blk, DE), lambda i: (i, 0)),
            pl.BlockSpec((blk, DE), lambda i: (i, 0)),
        ],
        out_shape=[jax.ShapeDtypeStruct((N, DE), jnp.float32)] * 2,
    )(x, wsd)


def _base_body(ea_ref, w_ref, b_ref, o_ref):
    o_ref[...] = (
        jnp.dot(ea_ref[...], w_ref[...], preferred_element_type=jnp.float32)
        + b_ref[...]
    )


def _tc_base(ea, wa, be2d):
    blk = E // 20
    return pl.pallas_call(
        _base_body,
        grid=(20,),
        in_specs=[
            pl.BlockSpec((blk, DE), lambda i: (i, 0)),
            pl.BlockSpec((DE, DE), lambda i: (0, 0)),
            pl.BlockSpec((1, DE), lambda i: (0, 0)),
        ],
        out_specs=pl.BlockSpec((blk, DE), lambda i: (i, 0)),
        out_shape=jax.ShapeDtypeStruct((E, DE), jnp.float32),
    )(ea, wa, be2d)


def _node_body(with_proj, x_ref, agg_ref, wnx_ref, wna_ref, bn_ref, wsd_ref,
               *out_refs):
    agg = agg_ref[0] + agg_ref[1]
    dx = (
        jnp.dot(x_ref[...], wnx_ref[...], preferred_element_type=jnp.float32)
        + jnp.dot(agg, wna_ref[...], preferred_element_type=jnp.float32)
        + bn_ref[...]
    )
    sa = jnp.float32(ALPHA) ** 0.5
    sb = jnp.float32(1.0 - ALPHA) ** 0.5
    xn = sa * jnp.maximum(dx, 0.0) + sb * x_ref[...]
    out_refs[0][...] = xn
    if with_proj:
        p = jnp.dot(xn, wsd_ref[...], preferred_element_type=jnp.float32)
        out_refs[1][...] = p[:, :DE]
        out_refs[2][...] = p[:, DE:]


def _tc_node(x, aggp, wnx, wna, bn2d, wsd_next, with_proj):
    blk = N // 10
    out_specs = [pl.BlockSpec((blk, D), lambda i: (i, 0))]
    out_shape = [jax.ShapeDtypeStruct((N, D), jnp.float32)]
    if with_proj:
        out_specs += [pl.BlockSpec((blk, DE), lambda i: (i, 0))] * 2
        out_shape += [jax.ShapeDtypeStruct((N, DE), jnp.float32)] * 2
    return pl.pallas_call(
        functools.partial(_node_body, with_proj),
        grid=(10,),
        in_specs=[
            pl.BlockSpec((blk, D), lambda i: (i, 0)),
            pl.BlockSpec((2, blk, DE), lambda i: (0, i, 0)),
            pl.BlockSpec((D, D), lambda i: (0, 0)),
            pl.BlockSpec((DE, D), lambda i: (0, 0)),
            pl.BlockSpec((1, D), lambda i: (0, 0)),
            pl.BlockSpec((D, 2 * DE), lambda i: (0, 0)),
        ],
        out_specs=out_specs,
        out_shape=out_shape,
    )(x, aggp, wnx, wna, bn2d, wsd_next)


def _sc_edge_body(ps_hbm, pd_hbm, base_hbm, eim_hbm, zeros_hbm,
                  ea_hbm, agg_hbm,
                  idx_s, idx_d,
                  rs0, rd0, bv0, ov0, rs1, rd1, bv1, ov1,
                  agg_sh, gs0, ss0, gs1, ss1):
    cid = lax.axis_index("c")
    sid = lax.axis_index("s")
    wid = sid * _NC + cid
    bufs = ((rs0, rd0, bv0, ov0, gs0, ss0), (rs1, rd1, bv1, ov1, gs1, ss1))

    pltpu.sync_copy(zeros_hbm.at[pl.ds(sid * _RPS, _RPS)],
                    agg_sh.at[pl.ds(sid * _RPS, _RPS)])
    pltpu.sync_copy(eim_hbm.at[0, wid], idx_s)
    pltpu.sync_copy(eim_hbm.at[1, wid], idx_d)
    plsc.subcore_barrier()

    def issue_gathers(j, b):
        rs, rd, bv, _, gs, _ = bufs[b]
        e0 = (wid * _KPW + j) * _CH
        return (
            pltpu.async_copy(ps_hbm.at[idx_s.at[j]], rs, gs),
            pltpu.async_copy(pd_hbm.at[idx_d.at[j]], rd, gs),
            pltpu.async_copy(base_hbm.at[pl.ds(e0, _CH)], bv, gs),
        )

    for j in range(_KPW):
        b = j % 2
        rs, rd, bv, ov, gs, ss = bufs[b]
        for dsc in issue_gathers(j, b):
            dsc.wait()

        def row(i, c, rs=rs, rd=rd, bv=bv, ov=ov):
            ov[i] = jnp.maximum(rs[i] + rd[i] + bv[i], 0.0)
            return c

        lax.fori_loop(0, _CH, row, None, unroll=10)

        e0 = (wid * _KPW + j) * _CH
        pltpu.sync_copy(ov, ea_hbm.at[pl.ds(e0, _CH)])
        pltpu.sync_copy(ov, agg_sh.at[idx_d.at[j]], add=True)

    plsc.subcore_barrier()
    pltpu.sync_copy(
        agg_sh.at[pl.ds(sid * _RPS, _RPS)],
        agg_hbm.at[pl.ds(cid * _N_PAD + sid * _RPS, _RPS)])


_sc_edge = functools.partial(
    pl.kernel,
    out_type=[
        jax.ShapeDtypeStruct((E, DE), jnp.float32),
        jax.ShapeDtypeStruct((2 * _N_PAD, DE), jnp.float32),
    ],
    mesh=plsc.VectorSubcoreMesh(core_axis_name="c", subcore_axis_name="s"),
    compiler_params=pltpu.CompilerParams(use_tc_tiling_on_sc=False),
    scratch_types=(
        [pltpu.VMEM((_KPW, _CH), jnp.int32)] * 2
        + [pltpu.VMEM((_CH, DE), jnp.float32)] * 8
        + [pltpu.VMEM_SHARED((_N_PAD, DE), jnp.float32)]
        + [pltpu.SemaphoreType.DMA] * 4
    ),
)(_sc_edge_body)


def kernel(x, edge_index, edge_attr, We1, be1, Wn1, bn1, We2, be2, Wn2, bn2):
    eim = edge_index.reshape(2, _NW, _KPW, _CH)
    zeros = jnp.zeros((_N_PAD, DE), jnp.float32)

    wsd1 = jnp.concatenate([We1[:D], We1[D:2 * D]], axis=1)
    wsd2 = jnp.concatenate([We2[:D], We2[D:2 * D]], axis=1)
    wa1, wa2 = We1[2 * D:], We2[2 * D:]
    be1_2d, be2_2d = be1[None, :], be2[None, :]
    wnx1, wna1 = Wn1[:D], Wn1[D:]
    wnx2, wna2 = Wn2[:D], Wn2[D:]
    bn1_2d, bn2_2d = bn1[None, :], bn2[None, :]

    ps1, pd1 = _tc_proj(x, wsd1)
    base1 = _tc_base(edge_attr, wa1, be1_2d)
    ea1, aggf1 = _sc_edge(ps1, pd1, base1, eim, zeros)
    aggp1 = aggf1.reshape(2, _N_PAD, DE)
    x2, ps2, pd2 = _tc_node(x, aggp1, wnx1, wna1, bn1_2d, wsd2, True)

    base2 = _tc_base(ea1, wa2, be2_2d)
    ea2, aggf2 = _sc_edge(ps2, pd2, base2, eim, zeros)
    aggp2 = aggf2.reshape(2, _N_PAD, DE)
    (x3,) = _tc_node(x2, aggp2, wnx2, wna2, bn2_2d, wsd2, False)

    return x3, ea2, jnp.concatenate([edge_attr, ea1, ea2], axis=1)


# speedup vs baseline: 4.2569x; 1.0831x over previous
import functools

import jax
import jax.numpy as jnp
from jax import lax
from jax.experimental import pallas as pl
from jax.experimental.pallas import tpu as pltpu
from jax.experimental.pallas import tpu_sc as plsc

N = 10000
E = 320000
D = 128
DE = 16
ALPHA = 0.5

_NC = 2
_NS = 16
_NW = _NC * _NS
_CH = 500
_KPW = 20
_N_PAD = 10240
_RPS = _N_PAD // _NS


def _proj_body(x_ref, w_ref, ps_ref, pd_ref):
    p = jnp.dot(x_ref[...], w_ref[...], preferred_element_type=jnp.float32)
    ps_ref[...] = p[:, :DE]
    pd_ref[...] = p[:, DE:]


def _tc_proj(x, wsd):
    blk = N // 10
    return pl.pallas_call(
        _proj_body,
        grid=(10,),
        in_specs=[
            pl.BlockSpec((blk, D), lambda i: (i, 0)),
            pl.BlockSpec((D, 2 * DE), lambda i: (0, 0)),
        ],
        out_specs=[
            pl.BlockSpec((blk, DE), lambda i: (i, 0)),
            pl.BlockSpec((blk, DE), lambda i: (i, 0)),
        ],
        out_shape=[jax.ShapeDtypeStruct((N, DE), jnp.float32)] * 2,
    )(x, wsd)


def _base_body(ea_ref, w_ref, b_ref, o_ref):
    o_ref[...] = (
        jnp.dot(ea_ref[...], w_ref[...], preferred_element_type=jnp.float32)
        + b_ref[...]
    )


def _tc_base(ea, wa, be2d):
    blk = E // 20
    return pl.pallas_call(
        _base_body,
        grid=(20,),
        in_specs=[
            pl.BlockSpec((blk, DE), lambda i: (i, 0)),
            pl.BlockSpec((DE, DE), lambda i: (0, 0)),
            pl.BlockSpec((1, DE), lambda i: (0, 0)),
        ],
        out_specs=pl.BlockSpec((blk, DE), lambda i: (i, 0)),
        out_shape=jax.ShapeDtypeStruct((E, DE), jnp.float32),
    )(ea, wa, be2d)


def _node_body(with_proj, x_ref, agg_ref, wnx_ref, wna_ref, bn_ref, wsd_ref,
               *out_refs):
    agg = agg_ref[0] + agg_ref[1]
    dx = (
        jnp.dot(x_ref[...], wnx_ref[...], preferred_element_type=jnp.float32)
        + jnp.dot(agg, wna_ref[...], preferred_element_type=jnp.float32)
        + bn_ref[...]
    )
    sa = jnp.float32(ALPHA) ** 0.5
    sb = jnp.float32(1.0 - ALPHA) ** 0.5
    xn = sa * jnp.maximum(dx, 0.0) + sb * x_ref[...]
    out_refs[0][...] = xn
    if with_proj:
        p = jnp.dot(xn, wsd_ref[...], preferred_element_type=jnp.float32)
        out_refs[1][...] = p[:, :DE]
        out_refs[2][...] = p[:, DE:]


def _tc_node(x, aggp, wnx, wna, bn2d, wsd_next, with_proj):
    blk = N // 10
    out_specs = [pl.BlockSpec((blk, D), lambda i: (i, 0))]
    out_shape = [jax.ShapeDtypeStruct((N, D), jnp.float32)]
    if with_proj:
        out_specs += [pl.BlockSpec((blk, DE), lambda i: (i, 0))] * 2
        out_shape += [jax.ShapeDtypeStruct((N, DE), jnp.float32)] * 2
    return pl.pallas_call(
        functools.partial(_node_body, with_proj),
        grid=(10,),
        in_specs=[
            pl.BlockSpec((blk, D), lambda i: (i, 0)),
            pl.BlockSpec((2, blk, DE), lambda i: (0, i, 0)),
            pl.BlockSpec((D, D), lambda i: (0, 0)),
            pl.BlockSpec((DE, D), lambda i: (0, 0)),
            pl.BlockSpec((1, D), lambda i: (0, 0)),
            pl.BlockSpec((D, 2 * DE), lambda i: (0, 0)),
        ],
        out_specs=out_specs,
        out_shape=out_shape,
    )(x, aggp, wnx, wna, bn2d, wsd_next)


def _sc_edge_body(ps_hbm, pd_hbm, base_hbm, eim_hbm, zeros_hbm,
                  ea_hbm, agg_hbm,
                  idx_s, idx_d,
                  rs0, rd0, bv0, ov0, rs1, rd1, bv1, ov1,
                  agg_sh, gs0, ss0, gs1, ss1):
    cid = lax.axis_index("c")
    sid = lax.axis_index("s")
    wid = sid * _NC + cid
    bufs = ((rs0, rd0, bv0, ov0, gs0, ss0), (rs1, rd1, bv1, ov1, gs1, ss1))

    pltpu.sync_copy(zeros_hbm.at[pl.ds(sid * _RPS, _RPS)],
                    agg_sh.at[pl.ds(sid * _RPS, _RPS)])
    pltpu.sync_copy(eim_hbm.at[0, wid], idx_s)
    pltpu.sync_copy(eim_hbm.at[1, wid], idx_d)
    plsc.subcore_barrier()

    def issue_gathers(j, b):
        rs, rd, bv, _, gs, _ = bufs[b]
        e0 = (wid * _KPW + j) * _CH
        return (
            pltpu.async_copy(ps_hbm.at[idx_s.at[j]], rs, gs),
            pltpu.async_copy(pd_hbm.at[idx_d.at[j]], rd, gs),
            pltpu.async_copy(base_hbm.at[pl.ds(e0, _CH)], bv, gs),
        )

    gd = [issue_gathers(0, 0), issue_gathers(1, 1)]
    sd = [None, None]
    for j in range(_KPW):
        b = j % 2
        rs, rd, bv, ov, gs, ss = bufs[b]
        for dsc in gd[b]:
            dsc.wait()
        if sd[b] is not None:
            sd[b].wait()

        def row(i, c, rs=rs, rd=rd, bv=bv, ov=ov):
            ov[i] = jnp.maximum(rs[i] + rd[i] + bv[i], 0.0)
            return c

        lax.fori_loop(0, _CH, row, None, unroll=10)

        e0 = (wid * _KPW + j) * _CH
        sd[b] = pltpu.async_copy(ov, ea_hbm.at[pl.ds(e0, _CH)], ss)
        pltpu.sync_copy(ov, agg_sh.at[idx_d.at[j]], add=True)
        if j + 2 < _KPW:
            gd[b] = issue_gathers(j + 2, b)

    for b in (0, 1):
        if sd[b] is not None:
            sd[b].wait()

    plsc.subcore_barrier()
    pltpu.sync_copy(
        agg_sh.at[pl.ds(sid * _RPS, _RPS)],
        agg_hbm.at[pl.ds(cid * _N_PAD + sid * _RPS, _RPS)])


_sc_edge = functools.partial(
    pl.kernel,
    out_type=[
        jax.ShapeDtypeStruct((E, DE), jnp.float32),
        jax.ShapeDtypeStruct((2 * _N_PAD, DE), jnp.float32),
    ],
    mesh=plsc.VectorSubcoreMesh(core_axis_name="c", subcore_axis_name="s"),
    compiler_params=pltpu.CompilerParams(use_tc_tiling_on_sc=False),
    scratch_types=(
        [pltpu.VMEM((_KPW, _CH), jnp.int32)] * 2
        + [pltpu.VMEM((_CH, DE), jnp.float32)] * 8
        + [pltpu.VMEM_SHARED((_N_PAD, DE), jnp.float32)]
        + [pltpu.SemaphoreType.DMA] * 4
    ),
)(_sc_edge_body)


def kernel(x, edge_index, edge_attr, We1, be1, Wn1, bn1, We2, be2, Wn2, bn2):
    eim = edge_index.reshape(2, _NW, _KPW, _CH)
    zeros = jnp.zeros((_N_PAD, DE), jnp.float32)

    wsd1 = jnp.concatenate([We1[:D], We1[D:2 * D]], axis=1)
    wsd2 = jnp.concatenate([We2[:D], We2[D:2 * D]], axis=1)
    wa1, wa2 = We1[2 * D:], We2[2 * D:]
    be1_2d, be2_2d = be1[None, :], be2[None, :]
    wnx1, wna1 = Wn1[:D], Wn1[D:]
    wnx2, wna2 = Wn2[:D], Wn2[D:]
    bn1_2d, bn2_2d = bn1[None, :], bn2[None, :]

    ps1, pd1 = _tc_proj(x, wsd1)
    base1 = _tc_base(edge_attr, wa1, be1_2d)
    ea1, aggf1 = _sc_edge(ps1, pd1, base1, eim, zeros)
    aggp1 = aggf1.reshape(2, _N_PAD, DE)
    x2, ps2, pd2 = _tc_node(x, aggp1, wnx1, wna1, bn1_2d, wsd2, True)

    base2 = _tc_base(ea1, wa2, be2_2d)
    ea2, aggf2 = _sc_edge(ps2, pd2, base2, eim, zeros)
    aggp2 = aggf2.reshape(2, _N_PAD, DE)
    (x3,) = _tc_node(x2, aggp2, wnx2, wna2, bn2_2d, wsd2, False)

    return x3, ea2, jnp.concatenate([edge_attr, ea1, ea2], axis=1)


# speedup vs baseline: 6.9617x; 1.6354x over previous
import functools

import jax
import jax.numpy as jnp
from jax import lax
from jax.experimental import pallas as pl
from jax.experimental.pallas import tpu as pltpu
from jax.experimental.pallas import tpu_sc as plsc

N = 10000
E = 320000
D = 128
DE = 16
ALPHA = 0.5

_NC = 2
_NS = 16
_NW = _NC * _NS
_CH = 400
_CHP = _CH // 8
_KPW = 25
_EP = E // 8
_N_PAD = 10240
_RPS = _N_PAD // _NS


def _proj_body(x_ref, w_ref, ps_ref, pd_ref):
    p = jnp.dot(x_ref[...], w_ref[...], preferred_element_type=jnp.float32)
    ps_ref[...] = p[:, :DE]
    pd_ref[...] = p[:, DE:]


def _tc_proj(x, wsd):
    blk = N // 10
    return pl.pallas_call(
        _proj_body,
        grid=(10,),
        in_specs=[
            pl.BlockSpec((blk, D), lambda i: (i, 0)),
            pl.BlockSpec((D, 2 * DE), lambda i: (0, 0)),
        ],
        out_specs=[
            pl.BlockSpec((blk, DE), lambda i: (i, 0)),
            pl.BlockSpec((blk, DE), lambda i: (i, 0)),
        ],
        out_shape=[jax.ShapeDtypeStruct((N, DE), jnp.float32)] * 2,
    )(x, wsd)


def _base_body(eap_ref, w8_ref, b8_ref, o_ref):
    o_ref[...] = (
        jnp.dot(eap_ref[...], w8_ref[...], preferred_element_type=jnp.float32)
        + b8_ref[...]
    )


def _tc_base(eap, w8, b8):
    blk = _EP // 20
    return pl.pallas_call(
        _base_body,
        grid=(20,),
        in_specs=[
            pl.BlockSpec((blk, D), lambda i: (i, 0)),
            pl.BlockSpec((D, D), lambda i: (0, 0)),
            pl.BlockSpec((1, D), lambda i: (0, 0)),
        ],
        out_specs=pl.BlockSpec((blk, D), lambda i: (i, 0)),
        out_shape=jax.ShapeDtypeStruct((_EP, D), jnp.float32),
    )(eap, w8, b8)


def _node_body(with_proj, x_ref, agg_ref, wnx_ref, wna_ref, bn_ref, wsd_ref,
               *out_refs):
    agg = agg_ref[0] + agg_ref[1]
    dx = (
        jnp.dot(x_ref[...], wnx_ref[...], preferred_element_type=jnp.float32)
        + jnp.dot(agg, wna_ref[...], preferred_element_type=jnp.float32)
        + bn_ref[...]
    )
    sa = jnp.float32(ALPHA) ** 0.5
    sb = jnp.float32(1.0 - ALPHA) ** 0.5
    xn = sa * jnp.maximum(dx, 0.0) + sb * x_ref[...]
    out_refs[0][...] = xn
    if with_proj:
        p = jnp.dot(xn, wsd_ref[...], preferred_element_type=jnp.float32)
        out_refs[1][...] = p[:, :DE]
        out_refs[2][...] = p[:, DE:]


def _tc_node(x, aggp, wnx, wna, bn2d, wsd_next, with_proj):
    blk = N // 10
    out_specs = [pl.BlockSpec((blk, D), lambda i: (i, 0))]
    out_shape = [jax.ShapeDtypeStruct((N, D), jnp.float32)]
    if with_proj:
        out_specs += [pl.BlockSpec((blk, DE), lambda i: (i, 0))] * 2
        out_shape += [jax.ShapeDtypeStruct((N, DE), jnp.float32)] * 2
    return pl.pallas_call(
        functools.partial(_node_body, with_proj),
        grid=(10,),
        in_specs=[
            pl.BlockSpec((blk, D), lambda i: (i, 0)),
            pl.BlockSpec((2, blk, DE), lambda i: (0, i, 0)),
            pl.BlockSpec((D, D), lambda i: (0, 0)),
            pl.BlockSpec((DE, D), lambda i: (0, 0)),
            pl.BlockSpec((1, D), lambda i: (0, 0)),
            pl.BlockSpec((D, 2 * DE), lambda i: (0, 0)),
        ],
        out_specs=out_specs,
        out_shape=out_shape,
    )(x, aggp, wnx, wna, bn2d, wsd_next)


def _sc_edge_body(ps_hbm, pd_hbm, base_hbm, eim_hbm, zeros_hbm,
                  ea_hbm, agg_hbm,
                  idx_s, idx_d,
                  rs0, rd0, bv0, ov0, os0, rs1, rd1, bv1, ov1, os1,
                  agg_sh, gs0, ss0, gs1, ss1):
    cid = lax.axis_index("c")
    sid = lax.axis_index("s")
    wid = sid * _NC + cid
    bufs = ((rs0, rd0, bv0, ov0, os0, gs0, ss0),
            (rs1, rd1, bv1, ov1, os1, gs1, ss1))

    pltpu.sync_copy(zeros_hbm.at[pl.ds(sid * _RPS, _RPS)],
                    agg_sh.at[pl.ds(sid * _RPS, _RPS)])
    pltpu.sync_copy(eim_hbm.at[0, wid], idx_s)
    pltpu.sync_copy(eim_hbm.at[1, wid], idx_d)
    plsc.subcore_barrier()

    def issue_gathers(j, b):
        rs, rd, bv, _, _, gs, _ = bufs[b]
        e8 = (wid * _KPW + j) * _CHP
        return (
            pltpu.async_copy(ps_hbm.at[idx_s.at[j]], rs, gs),
            pltpu.async_copy(pd_hbm.at[idx_d.at[j]], rd, gs),
            pltpu.async_copy(base_hbm.at[pl.ds(e8, _CHP)], bv, gs),
        )

    gd = [issue_gathers(0, 0), issue_gathers(1, 1)]
    sd = [None, None]
    for j in range(_KPW):
        b = j % 2
        rs, rd, bv, ov, os_, gs, ss = bufs[b]
        for dsc in gd[b]:
            dsc.wait()
        if sd[b] is not None:
            sd[b].wait()

        def row(r, c_, rs=rs, rd=rd, bv=bv, ov=ov, os_=os_):
            for c in range(8):
                i = r * 8 + c
                v = jnp.maximum(
                    rs[i] + rd[i] + bv[r, pl.ds(c * DE, DE)], 0.0)
                ov[r, pl.ds(c * DE, DE)] = v
                os_[i] = v
            return c_

        lax.fori_loop(0, _CHP, row, None, unroll=2)

        e8 = (wid * _KPW + j) * _CHP
        sd[b] = pltpu.async_copy(ov, ea_hbm.at[pl.ds(e8, _CHP)], ss)
        pltpu.sync_copy(os_, agg_sh.at[idx_d.at[j]], add=True)
        if j + 2 < _KPW:
            gd[b] = issue_gathers(j + 2, b)

    for b in (0, 1):
        if sd[b] is not None:
            sd[b].wait()

    plsc.subcore_barrier()
    pltpu.sync_copy(
        agg_sh.at[pl.ds(sid * _RPS, _RPS)],
        agg_hbm.at[pl.ds(cid * _N_PAD + sid * _RPS, _RPS)])


_sc_edge = functools.partial(
    pl.kernel,
    out_type=[
        jax.ShapeDtypeStruct((_EP, D), jnp.float32),
        jax.ShapeDtypeStruct((2 * _N_PAD, DE), jnp.float32),
    ],
    mesh=plsc.VectorSubcoreMesh(core_axis_name="c", subcore_axis_name="s"),
    compiler_params=pltpu.CompilerParams(use_tc_tiling_on_sc=False),
    scratch_types=(
        [pltpu.VMEM((_KPW, _CH), jnp.int32)] * 2
        + [pltpu.VMEM((_CH, DE), jnp.float32),
           pltpu.VMEM((_CH, DE), jnp.float32),
           pltpu.VMEM((_CHP, D), jnp.float32),
           pltpu.VMEM((_CHP, D), jnp.float32),
           pltpu.VMEM((_CH, DE), jnp.float32)] * 2
        + [pltpu.VMEM_SHARED((_N_PAD, DE), jnp.float32)]
        + [pltpu.SemaphoreType.DMA] * 4
    ),
)(_sc_edge_body)


def kernel(x, edge_index, edge_attr, We1, be1, Wn1, bn1, We2, be2, Wn2, bn2):
    eim = edge_index.reshape(2, _NW, _KPW, _CH)
    zeros = jnp.zeros((_N_PAD, DE), jnp.float32)
    eye8 = jnp.eye(8, dtype=jnp.float32)

    wsd1 = jnp.concatenate([We1[:D], We1[D:2 * D]], axis=1)
    wsd2 = jnp.concatenate([We2[:D], We2[D:2 * D]], axis=1)
    w8_1 = jnp.kron(eye8, We1[2 * D:])
    w8_2 = jnp.kron(eye8, We2[2 * D:])
    b8_1 = jnp.tile(be1, 8)[None, :]
    b8_2 = jnp.tile(be2, 8)[None, :]
    wnx1, wna1 = Wn1[:D], Wn1[D:]
    wnx2, wna2 = Wn2[:D], Wn2[D:]
    bn1_2d, bn2_2d = bn1[None, :], bn2[None, :]

    ps1, pd1 = _tc_proj(x, wsd1)
    base1 = _tc_base(edge_attr.reshape(_EP, D), w8_1, b8_1)
    ea1p, aggf1 = _sc_edge(ps1, pd1, base1, eim, zeros)
    aggp1 = aggf1.reshape(2, _N_PAD, DE)
    x2, ps2, pd2 = _tc_node(x, aggp1, wnx1, wna1, bn1_2d, wsd2, True)

    base2 = _tc_base(ea1p, w8_2, b8_2)
    ea2p, aggf2 = _sc_edge(ps2, pd2, base2, eim, zeros)
    aggp2 = aggf2.reshape(2, _N_PAD, DE)
    (x3,) = _tc_node(x2, aggp2, wnx2, wna2, bn2_2d, wsd2, False)

    ea1 = ea1p.reshape(E, DE)
    ea2 = ea2p.reshape(E, DE)
    return x3, ea2, jnp.concatenate([edge_attr, ea1, ea2], axis=1)
